# baseline (speedup 1.0000x reference)
import functools
import math

import jax
import jax.numpy as jnp
from jax import lax
from jax.experimental import pallas as pl
from jax.experimental.pallas import tpu as pltpu
from jax.experimental.pallas import tpu_sc as plsc

DIM = 32
SCALE = math.sqrt(32.0)

_NC = 2
_NS = 16
_NW = _NC * _NS

_S = 4096
_T = 200
_SB = 32
_TPG = 4
_GSZ = _TPG * 128
_NG = _T // _TPG
_NTG = _T // 8


def _transpose_scale(rows, ob, iota):
    for t_local in range(_TPG):
        def cbody(c, carry, t_local=t_local):
            idxc = jnp.full((16,), 0, jnp.int32) + c
            for j0 in range(8):
                idx0 = t_local * 128 + j0 * 16 + iota
                v = plsc.load_gather(rows, [idx0, idxc])
                ob[t_local, c // 8, c % 8, pl.ds(j0 * 16, 16)] = v * SCALE
            return carry

        lax.fori_loop(0, DIM, cbody, 0)


def _emb_kernel(idx_hbm, table_hbm, out_hbm, idx_all, rows0, rows1,
                ob0, ob1, isem, gsem0, gsem1, osem0, osem1):
    rows = (rows0, rows1)
    ob = (ob0, ob1)
    gsem = (gsem0, gsem1)
    osem = (osem0, osem1)
    wid = lax.axis_index("s") * _NC + lax.axis_index("c")
    iota = lax.iota(jnp.int32, 16)

    idx_copies = [
        pltpu.async_copy(idx_hbm.at[i, wid], idx_all.at[i], isem)
        for i in range(_NTG)
    ]
    for c in idx_copies:
        c.wait()

    def start_gather(g, b):
        tg = g // 2
        half = g % 2
        return pltpu.async_copy(
            table_hbm.at[idx_all.at[tg, pl.ds(half * _GSZ, _GSZ)]],
            rows[b], gsem[b],
        )

    def start_outs(g, b):
        for tl in range(_TPG):
            pltpu.async_copy(
                ob[b].at[tl], out_hbm.at[g * _TPG + tl, :, wid], osem[b]
            )

    def wait_outs(g, b):
        for tl in range(_TPG):
            pltpu.make_async_copy(
                ob[b].at[tl], out_hbm.at[g * _TPG + tl, :, wid], osem[b]
            ).wait()

    start_gather(0, 0)
    start_gather(1, 1)

    def gbody(gbase, carry):
        for b in range(2):
            g = gbase + b
            pltpu.make_async_copy(
                table_hbm.at[idx_all.at[g // 2, pl.ds((g % 2) * _GSZ, _GSZ)]],
                rows[b], gsem[b],
            ).wait()

            @pl.when(gbase >= 2)
            def _():
                wait_outs(g - 2, b)

            _transpose_scale(rows[b], ob[b], iota)

            @pl.when(gbase + 2 < _NG)
            def _():
                start_gather(g + 2, b)

            start_outs(g, b)
        return carry

    lax.fori_loop(0, _NG // 2, lambda i, c: gbody(i * 2, c), 0)
    wait_outs(_NG - 2, 0)
    wait_outs(_NG - 1, 1)


@functools.partial(
    pl.kernel,
    mesh=plsc.VectorSubcoreMesh(core_axis_name="c", subcore_axis_name="s"),
    out_type=jax.ShapeDtypeStruct((_T, DIM // 8, _SB, 8, 128), jnp.float32),
    scratch_types=[
        pltpu.VMEM((_NTG, 1024), jnp.int32),
        pltpu.VMEM((_GSZ, DIM), jnp.float32),
        pltpu.VMEM((_GSZ, DIM), jnp.float32),
        pltpu.VMEM((_TPG, DIM // 8, 8, 128), jnp.float32),
        pltpu.VMEM((_TPG, DIM // 8, 8, 128), jnp.float32),
        pltpu.SemaphoreType.DMA,
        pltpu.SemaphoreType.DMA,
        pltpu.SemaphoreType.DMA,
        pltpu.SemaphoreType.DMA,
        pltpu.SemaphoreType.DMA,
    ],
    compiler_params=pltpu.CompilerParams(
        use_tc_tiling_on_sc=False, needs_layout_passes=False
    ),
)
def _emb(idx_hbm, table_hbm, out_hbm, idx_all, rows0, rows1, ob0, ob1,
         isem, gsem0, gsem1, osem0, osem1):
    _emb_kernel(idx_hbm, table_hbm, out_hbm, idx_all, rows0, rows1,
                ob0, ob1, isem, gsem0, gsem1, osem0, osem1)


@jax.jit
def kernel(tokens, table):
    idx = (
        tokens.T.astype(jnp.int32)
        .reshape(_NTG, 8, _SB, 128)
        .transpose(0, 2, 1, 3)
        .reshape(_NTG, _SB, 1024)
    )
    out5 = _emb(idx, table)
    return (
        out5.transpose(2, 4, 0, 1, 3)
        .reshape(_S, _T, DIM)
    )


# speedup vs baseline: 1.4237x; 1.4237x over previous
import functools
import math

import jax
import jax.numpy as jnp
from jax import lax
from jax.experimental import pallas as pl
from jax.experimental.pallas import tpu as pltpu
from jax.experimental.pallas import tpu_sc as plsc

DIM = 32
SCALE = math.sqrt(32.0)

_NC = 2
_NS = 16
_NW = _NC * _NS

_S = 4096
_T = 200
_SB = 32
_TPG = 4
_GSZ = _TPG * 128
_NG = _T // _TPG
_NTG = _T // 8


def _transpose_scale(rows, ob, iota, perms):
    for t_local in range(_TPG):
        def jbody(j, carry, t_local=t_local):
            j0 = j * 16
            row_idx = t_local * 128 + j0 + iota
            for c0 in (0, 16):
                for k in range(16):
                    perm, st_off = perms[k]
                    col_idx = perm + c0
                    v = plsc.load_gather(rows, [row_idx, col_idx])
                    dst = st_off + (t_local * 4 + c0 // 8) * 1024 + j0
                    plsc.store_scatter(ob, [dst], v * SCALE)
            return carry

        lax.fori_loop(0, 8, jbody, 0)


def _emb_kernel(idx_hbm, table_hbm, out_hbm, idx_all, rows0, rows1,
                ob0, ob1, isem, gsem0, gsem1, osem0, osem1):
    rows = (rows0, rows1)
    ob = (ob0, ob1)
    gsem = (gsem0, gsem1)
    osem = (osem0, osem1)
    wid = lax.axis_index("s") * _NC + lax.axis_index("c")
    iota = lax.iota(jnp.int32, 16)
    perms = []
    for k in range(16):
        perm = (iota + k) & 15
        st_off = (perm >> 3) * 1024 + (perm & 7) * 128 + iota
        perms.append((perm, st_off))

    idx_copies = [
        pltpu.async_copy(idx_hbm.at[i, wid], idx_all.at[i], isem)
        for i in range(_NTG)
    ]
    for c in idx_copies:
        c.wait()

    def gather_copy(g, b):
        return pltpu.make_async_copy(
            table_hbm.at[idx_all.at[g // 2, pl.ds((g % 2) * _GSZ, _GSZ)]],
            rows[b], gsem[b],
        )

    def out_copies(g, b):
        res = []
        for tl in range(_TPG):
            for cg in range(4):
                res.append(pltpu.make_async_copy(
                    ob[b].at[pl.ds((tl * 4 + cg) * 1024, 1024)],
                    out_hbm.at[g * _TPG + tl, cg, wid],
                    osem[b],
                ))
        return res

    gather_copy(0, 0).start()
    gather_copy(1, 1).start()

    def gbody(gbase, carry):
        for b in range(2):
            g = gbase + b
            gather_copy(g, b).wait()

            @pl.when(gbase >= 2)
            def _():
                for c in out_copies(g - 2, b):
                    c.wait()

            _transpose_scale(rows[b], ob[b], iota, perms)

            @pl.when(gbase + 2 < _NG)
            def _():
                gather_copy(g + 2, b).start()

            for c in out_copies(g, b):
                c.start()
        return carry

    lax.fori_loop(0, _NG // 2, lambda i, c: gbody(i * 2, c), 0)
    for c in out_copies(_NG - 2, 0):
        c.wait()
    for c in out_copies(_NG - 1, 1):
        c.wait()


@functools.partial(
    pl.kernel,
    mesh=plsc.VectorSubcoreMesh(core_axis_name="c", subcore_axis_name="s"),
    out_type=jax.ShapeDtypeStruct((_T, DIM // 8, _SB, 1024), jnp.float32),
    scratch_types=[
        pltpu.VMEM((_NTG, 1024), jnp.int32),
        pltpu.VMEM((_GSZ, DIM), jnp.float32),
        pltpu.VMEM((_GSZ, DIM), jnp.float32),
        pltpu.VMEM((_TPG * DIM // 8 * 1024,), jnp.float32),
        pltpu.VMEM((_TPG * DIM // 8 * 1024,), jnp.float32),
        pltpu.SemaphoreType.DMA,
        pltpu.SemaphoreType.DMA,
        pltpu.SemaphoreType.DMA,
        pltpu.SemaphoreType.DMA,
        pltpu.SemaphoreType.DMA,
    ],
    compiler_params=pltpu.CompilerParams(
        use_tc_tiling_on_sc=False, needs_layout_passes=False
    ),
)
def _emb(idx_hbm, table_hbm, out_hbm, idx_all, rows0, rows1, ob0, ob1,
         isem, gsem0, gsem1, osem0, osem1):
    _emb_kernel(idx_hbm, table_hbm, out_hbm, idx_all, rows0, rows1,
                ob0, ob1, isem, gsem0, gsem1, osem0, osem1)


@jax.jit
def kernel(tokens, table):
    idx = (
        tokens.T.astype(jnp.int32)
        .reshape(_NTG, 8, _SB, 128)
        .transpose(0, 2, 1, 3)
        .reshape(_NTG, _SB, 1024)
    )
    out5 = _emb(idx, table)
    return (
        out5.reshape(_T, DIM // 8, _SB, 8, 128)
        .transpose(2, 4, 0, 1, 3)
        .reshape(_S, _T, DIM)
    )


# speedup vs baseline: 1.5419x; 1.0830x over previous
import functools
import math

import jax
import jax.numpy as jnp
from jax import lax
from jax.experimental import pallas as pl
from jax.experimental.pallas import tpu as pltpu
from jax.experimental.pallas import tpu_sc as plsc

DIM = 32
SCALE = math.sqrt(32.0)

_NC = 2
_NS = 16
_NW = _NC * _NS

_S = 4096
_T = 200
_SB = 32
_TPG = 4
_GSZ = _TPG * 128
_NG = _T // _TPG
_NTG = _T // 8


def _transpose_scale(rows, ob, iota, perms):
    for t_local in range(_TPG):
        @plsc.parallel_loop(0, 8)
        def jbody(j, t_local=t_local):
            j0 = j * 16
            row_idx = t_local * 128 + j0 + iota
            for c0 in (0, 16):
                for k in range(16):
                    perm, st_off = perms[k]
                    col_idx = perm + c0
                    v = plsc.load_gather(rows, [row_idx, col_idx])
                    dst = st_off + (t_local * 4 + c0 // 8) * 1024 + j0
                    plsc.store_scatter(ob, [dst], v * SCALE)


def _emb_kernel(idx_hbm, table_hbm, out_hbm, idx_all, rows0, rows1,
                ob0, ob1, isem, gsem0, gsem1, osem0, osem1):
    rows = (rows0, rows1)
    ob = (ob0, ob1)
    gsem = (gsem0, gsem1)
    osem = (osem0, osem1)
    wid = lax.axis_index("s") * _NC + lax.axis_index("c")
    iota = lax.iota(jnp.int32, 16)
    perms = []
    for k in range(16):
        perm = (iota + k) & 15
        st_off = (perm >> 3) * 1024 + (perm & 7) * 128 + iota
        perms.append((perm, st_off))

    idx_copies = [
        pltpu.async_copy(idx_hbm.at[i, wid], idx_all.at[i], isem)
        for i in range(_NTG)
    ]
    for c in idx_copies:
        c.wait()

    def gather_copy(g, b):
        return pltpu.make_async_copy(
            table_hbm.at[idx_all.at[g // 2, pl.ds((g % 2) * _GSZ, _GSZ)]],
            rows[b], gsem[b],
        )

    def out_copies(g, b):
        res = []
        for tl in range(_TPG):
            for cg in range(4):
                res.append(pltpu.make_async_copy(
                    ob[b].at[pl.ds((tl * 4 + cg) * 1024, 1024)],
                    out_hbm.at[g * _TPG + tl, cg, wid],
                    osem[b],
                ))
        return res

    gather_copy(0, 0).start()
    gather_copy(1, 1).start()

    def gbody(gbase, carry):
        for b in range(2):
            g = gbase + b
            gather_copy(g, b).wait()

            @pl.when(gbase >= 2)
            def _():
                for c in out_copies(g - 2, b):
                    c.wait()

            _transpose_scale(rows[b], ob[b], iota, perms)

            @pl.when(gbase + 2 < _NG)
            def _():
                gather_copy(g + 2, b).start()

            for c in out_copies(g, b):
                c.start()
        return carry

    lax.fori_loop(0, _NG // 2, lambda i, c: gbody(i * 2, c), 0)
    for c in out_copies(_NG - 2, 0):
        c.wait()
    for c in out_copies(_NG - 1, 1):
        c.wait()


@functools.partial(
    pl.kernel,
    mesh=plsc.VectorSubcoreMesh(core_axis_name="c", subcore_axis_name="s"),
    out_type=jax.ShapeDtypeStruct((_T, DIM // 8, _SB, 1024), jnp.float32),
    scratch_types=[
        pltpu.VMEM((_NTG, 1024), jnp.int32),
        pltpu.VMEM((_GSZ, DIM), jnp.float32),
        pltpu.VMEM((_GSZ, DIM), jnp.float32),
        pltpu.VMEM((_TPG * DIM // 8 * 1024,), jnp.float32),
        pltpu.VMEM((_TPG * DIM // 8 * 1024,), jnp.float32),
        pltpu.SemaphoreType.DMA,
        pltpu.SemaphoreType.DMA,
        pltpu.SemaphoreType.DMA,
        pltpu.SemaphoreType.DMA,
        pltpu.SemaphoreType.DMA,
    ],
    compiler_params=pltpu.CompilerParams(
        use_tc_tiling_on_sc=False, needs_layout_passes=False
    ),
)
def _emb(idx_hbm, table_hbm, out_hbm, idx_all, rows0, rows1, ob0, ob1,
         isem, gsem0, gsem1, osem0, osem1):
    _emb_kernel(idx_hbm, table_hbm, out_hbm, idx_all, rows0, rows1,
                ob0, ob1, isem, gsem0, gsem1, osem0, osem1)


@jax.jit
def kernel(tokens, table):
    idx = (
        tokens.T.astype(jnp.int32)
        .reshape(_NTG, 8, _SB, 128)
        .transpose(0, 2, 1, 3)
        .reshape(_NTG, _SB, 1024)
    )
    out5 = _emb(idx, table)
    return (
        out5.reshape(_T, DIM // 8, _SB, 8, 128)
        .transpose(2, 4, 0, 1, 3)
        .reshape(_S, _T, DIM)
    )


# speedup vs baseline: 1.8569x; 1.2043x over previous
import functools
import math

import jax
import jax.numpy as jnp
from jax import lax
from jax.experimental import pallas as pl
from jax.experimental.pallas import tpu as pltpu
from jax.experimental.pallas import tpu_sc as plsc

DIM = 32
SCALE = math.sqrt(32.0)

_NC = 2
_NS = 16
_NW = _NC * _NS

_S = 4096
_T = 200
_SB = 32
_TPG = 4
_GSZ = _TPG * 128
_NG = _T // _TPG
_NTG = _T // 8


def _transpose_scale(rows, ob, iota, perms):
    @plsc.parallel_loop(0, _TPG * 8)
    def jbody(i):
        t_local = i >> 3
        j0 = (i & 7) * 16
        row_idx = t_local * 128 + j0 + iota
        base = t_local * 4096 + j0
        for c0 in (0, 16):
            for k in range(16):
                perm, st_off = perms[k]
                col_idx = perm + c0
                v = plsc.load_gather(rows, [row_idx, col_idx])
                dst = st_off + (c0 // 8) * 1024 + base
                plsc.store_scatter(ob, [dst], v * SCALE)


def _emb_kernel(idx_hbm, table_hbm, out_hbm, idx_all, rows0, rows1,
                ob0, ob1, isem, gsem0, gsem1, osem0, osem1):
    rows = (rows0, rows1)
    ob = (ob0, ob1)
    gsem = (gsem0, gsem1)
    osem = (osem0, osem1)
    wid = lax.axis_index("s") * _NC + lax.axis_index("c")
    iota = lax.iota(jnp.int32, 16)
    perms = []
    for k in range(16):
        perm = (iota + k) & 15
        st_off = (perm >> 3) * 1024 + (perm & 7) * 128 + iota
        perms.append((perm, st_off))

    idx_copies = [
        pltpu.async_copy(idx_hbm.at[i, wid], idx_all.at[i], isem)
        for i in range(_NTG)
    ]
    for c in idx_copies:
        c.wait()

    def gather_copy(g, b):
        return pltpu.make_async_copy(
            table_hbm.at[idx_all.at[g // 2, pl.ds((g % 2) * _GSZ, _GSZ)]],
            rows[b], gsem[b],
        )

    def out_copies(g, b):
        res = []
        for tl in range(_TPG):
            for cg in range(4):
                res.append(pltpu.make_async_copy(
                    ob[b].at[pl.ds((tl * 4 + cg) * 1024, 1024)],
                    out_hbm.at[g * _TPG + tl, cg, wid],
                    osem[b],
                ))
        return res

    gather_copy(0, 0).start()
    gather_copy(1, 1).start()

    def gbody(gbase, carry):
        for b in range(2):
            g = gbase + b
            gather_copy(g, b).wait()

            @pl.when(gbase >= 2)
            def _():
                for c in out_copies(g - 2, b):
                    c.wait()

            _transpose_scale(rows[b], ob[b], iota, perms)

            @pl.when(gbase + 2 < _NG)
            def _():
                gather_copy(g + 2, b).start()

            for c in out_copies(g, b):
                c.start()
        return carry

    lax.fori_loop(0, _NG // 2, lambda i, c: gbody(i * 2, c), 0)
    for c in out_copies(_NG - 2, 0):
        c.wait()
    for c in out_copies(_NG - 1, 1):
        c.wait()


@functools.partial(
    pl.kernel,
    mesh=plsc.VectorSubcoreMesh(core_axis_name="c", subcore_axis_name="s"),
    out_type=jax.ShapeDtypeStruct((_T, DIM // 8, _SB, 1024), jnp.float32),
    scratch_types=[
        pltpu.VMEM((_NTG, 1024), jnp.int32),
        pltpu.VMEM((_GSZ, DIM), jnp.float32),
        pltpu.VMEM((_GSZ, DIM), jnp.float32),
        pltpu.VMEM((_TPG * DIM // 8 * 1024,), jnp.float32),
        pltpu.VMEM((_TPG * DIM // 8 * 1024,), jnp.float32),
        pltpu.SemaphoreType.DMA,
        pltpu.SemaphoreType.DMA,
        pltpu.SemaphoreType.DMA,
        pltpu.SemaphoreType.DMA,
        pltpu.SemaphoreType.DMA,
    ],
    compiler_params=pltpu.CompilerParams(
        use_tc_tiling_on_sc=False, needs_layout_passes=False
    ),
)
def _emb(idx_hbm, table_hbm, out_hbm, idx_all, rows0, rows1, ob0, ob1,
         isem, gsem0, gsem1, osem0, osem1):
    _emb_kernel(idx_hbm, table_hbm, out_hbm, idx_all, rows0, rows1,
                ob0, ob1, isem, gsem0, gsem1, osem0, osem1)


@jax.jit
def kernel(tokens, table):
    idx = (
        tokens.T.astype(jnp.int32)
        .reshape(_NTG, 8, _SB, 128)
        .transpose(0, 2, 1, 3)
        .reshape(_NTG, _SB, 1024)
    )
    out5 = _emb(idx, table)
    return (
        out5.reshape(_T, DIM // 8, _SB, 8, 128)
        .transpose(2, 4, 0, 1, 3)
        .reshape(_S, _T, DIM)
    )


# speedup vs baseline: 1.9155x; 1.0316x over previous
import functools
import math

import jax
import jax.numpy as jnp
from jax import lax
from jax.experimental import pallas as pl
from jax.experimental.pallas import tpu as pltpu
from jax.experimental.pallas import tpu_sc as plsc

DIM = 32
SCALE = math.sqrt(32.0)

_NC = 2
_NS = 16
_NW = _NC * _NS

_S = 4096
_T = 200
_SB = 32
_TPG = 4
_GSZ = _TPG * 128
_NG = _T // _TPG
_NTG = _T // 8


def _transpose_scale(rows, ob, iota, perms):
    @plsc.parallel_loop(0, _TPG * 8, unroll=2)
    def jbody(i):
        t_local = i >> 3
        j0 = (i & 7) * 16
        row_idx = t_local * 128 + j0 + iota
        base = t_local * 4096 + j0
        for c0 in (0, 16):
            for k in range(16):
                perm, st_off = perms[k]
                col_idx = perm + c0
                v = plsc.load_gather(rows, [row_idx, col_idx])
                dst = st_off + (c0 // 8) * 1024 + base
                plsc.store_scatter(ob, [dst], v * SCALE)


def _emb_kernel(idx_hbm, table_hbm, out_hbm, idx_all, rows0, rows1,
                ob0, ob1, isem, gsem0, gsem1, osem0, osem1):
    rows = (rows0, rows1)
    ob = (ob0, ob1)
    gsem = (gsem0, gsem1)
    osem = (osem0, osem1)
    wid = lax.axis_index("s") * _NC + lax.axis_index("c")
    iota = lax.iota(jnp.int32, 16)
    perms = []
    for k in range(16):
        perm = (iota + k) & 15
        st_off = (perm >> 3) * 1024 + (perm & 7) * 128 + iota
        perms.append((perm, st_off))

    idx_copies = [
        pltpu.async_copy(idx_hbm.at[i, wid], idx_all.at[i], isem)
        for i in range(_NTG)
    ]
    for c in idx_copies:
        c.wait()

    def gather_copy(g, b):
        return pltpu.make_async_copy(
            table_hbm.at[idx_all.at[g // 2, pl.ds((g % 2) * _GSZ, _GSZ)]],
            rows[b], gsem[b],
        )

    def out_copies(g, b):
        res = []
        for tl in range(_TPG):
            for cg in range(4):
                res.append(pltpu.make_async_copy(
                    ob[b].at[pl.ds((tl * 4 + cg) * 1024, 1024)],
                    out_hbm.at[g * _TPG + tl, cg, wid],
                    osem[b],
                ))
        return res

    gather_copy(0, 0).start()
    gather_copy(1, 1).start()

    def gbody(gbase, carry):
        for b in range(2):
            g = gbase + b
            gather_copy(g, b).wait()

            @pl.when(gbase >= 2)
            def _():
                for c in out_copies(g - 2, b):
                    c.wait()

            _transpose_scale(rows[b], ob[b], iota, perms)

            @pl.when(gbase + 2 < _NG)
            def _():
                gather_copy(g + 2, b).start()

            for c in out_copies(g, b):
                c.start()
        return carry

    lax.fori_loop(0, _NG // 2, lambda i, c: gbody(i * 2, c), 0)
    for c in out_copies(_NG - 2, 0):
        c.wait()
    for c in out_copies(_NG - 1, 1):
        c.wait()


@functools.partial(
    pl.kernel,
    mesh=plsc.VectorSubcoreMesh(core_axis_name="c", subcore_axis_name="s"),
    out_type=jax.ShapeDtypeStruct((_T, DIM // 8, _SB, 1024), jnp.float32),
    scratch_types=[
        pltpu.VMEM((_NTG, 1024), jnp.int32),
        pltpu.VMEM((_GSZ, DIM), jnp.float32),
        pltpu.VMEM((_GSZ, DIM), jnp.float32),
        pltpu.VMEM((_TPG * DIM // 8 * 1024,), jnp.float32),
        pltpu.VMEM((_TPG * DIM // 8 * 1024,), jnp.float32),
        pltpu.SemaphoreType.DMA,
        pltpu.SemaphoreType.DMA,
        pltpu.SemaphoreType.DMA,
        pltpu.SemaphoreType.DMA,
        pltpu.SemaphoreType.DMA,
    ],
    compiler_params=pltpu.CompilerParams(
        use_tc_tiling_on_sc=False, needs_layout_passes=False
    ),
)
def _emb(idx_hbm, table_hbm, out_hbm, idx_all, rows0, rows1, ob0, ob1,
         isem, gsem0, gsem1, osem0, osem1):
    _emb_kernel(idx_hbm, table_hbm, out_hbm, idx_all, rows0, rows1,
                ob0, ob1, isem, gsem0, gsem1, osem0, osem1)


@jax.jit
def kernel(tokens, table):
    idx = (
        tokens.T.astype(jnp.int32)
        .reshape(_NTG, 8, _SB, 128)
        .transpose(0, 2, 1, 3)
        .reshape(_NTG, _SB, 1024)
    )
    out5 = _emb(idx, table)
    return (
        out5.reshape(_T, DIM // 8, _SB, 8, 128)
        .transpose(2, 4, 0, 1, 3)
        .reshape(_S, _T, DIM)
    )
